# baseline (speedup 1.0000x reference)
import jax
import jax.numpy as jnp
from jax.experimental import pallas as pl
from jax.experimental.pallas import tpu as pltpu

THRES_STATIC = 0.001
THRES_DIST = 0.002

B, N, M = 4, 4096, 4096
MT = 4096


def _dist2_body(x_ref, w_ref, y_ref, y8_ref, out_ref):
    a = x_ref[0]
    yb = y_ref[0]
    aa = jnp.sum(a * a, axis=1, keepdims=True)
    ab = jax.lax.dot_general(
        w_ref[0], y8_ref[0],
        (((1,), (0,)), ((), ())),
        preferred_element_type=jnp.float32,
    )
    t = ab + aa
    tmin = jnp.min(t, axis=0, keepdims=True)
    bb = jnp.sum(yb * yb, axis=0, keepdims=True)
    out_ref[...] = jnp.maximum(tmin + bb, 0.0)[None]


def _loss_body(d2_ref, y0_ref, y1_ref, sdyn_ref, cdyn_ref, sst_ref, cst_ref):
    d2 = d2_ref[...]
    pos = d2 < THRES_DIST
    neg = d2 > THRES_STATIC
    sq = jnp.zeros_like(d2)
    sst = jnp.float32(0.0)
    cst = jnp.float32(0.0)
    for c in range(3):
        diff = y1_ref[c * B:(c + 1) * B, :] - y0_ref[c * B:(c + 1) * B, :]
        sq = sq + diff * diff
        sel_st = jnp.logical_and(neg, diff > 0.0)
        sst = sst + jnp.sum(jnp.where(sel_st, diff * diff, 0.0))
        cst = cst + jnp.sum(sel_st.astype(jnp.float32))
    sel_dyn = jnp.logical_and(pos, sq > 0.0)
    norm = jnp.sqrt(jnp.where(sel_dyn, sq, 1.0))
    z = 0.1 - norm
    sp = jnp.maximum(z, 0.0) + jnp.log1p(jnp.exp(-jnp.abs(z)))
    sdyn_ref[0, 0] = jnp.sum(jnp.where(sel_dyn, sp, 0.0))
    cdyn_ref[0, 0] = jnp.sum(sel_dyn.astype(jnp.float32))
    sst_ref[0, 0] = sst
    cst_ref[0, 0] = cst


@jax.jit
def kernel(x, y_hat0, y_hat1):
    xp = x[:, 0, :, :3].astype(jnp.float32)
    y0p = y_hat0[:, 0].astype(jnp.float32)
    yt = jnp.transpose(y0p, (0, 2, 1))
    wp = jnp.zeros((B, N, 8), jnp.bfloat16).at[:, :, :3].set(
        (-2.0 * xp).astype(jnp.bfloat16))
    y8 = jnp.zeros((B, 8, M), jnp.bfloat16).at[:, :3, :].set(
        yt.astype(jnp.bfloat16))

    d2 = pl.pallas_call(
        _dist2_body,
        grid=(B, M // MT),
        in_specs=[
            pl.BlockSpec((1, N, 3), lambda b, m: (b, 0, 0)),
            pl.BlockSpec((1, N, 8), lambda b, m: (b, 0, 0)),
            pl.BlockSpec((1, 3, MT), lambda b, m: (b, 0, m)),
            pl.BlockSpec((1, 8, MT), lambda b, m: (b, 0, m)),
        ],
        out_specs=pl.BlockSpec((1, 1, MT), lambda b, m: (b, 0, m)),
        out_shape=jax.ShapeDtypeStruct((B, 1, M), jnp.float32),
    )(xp, wp, yt, y8)
    d2 = d2[:, 0, :]

    y0r = jnp.transpose(y0p, (2, 0, 1)).reshape(3 * B, M)
    y1r = jnp.transpose(y_hat1[:, 0].astype(jnp.float32), (2, 0, 1)).reshape(3 * B, M)

    scal = jax.ShapeDtypeStruct((1, 1), jnp.float32)
    sspec = pl.BlockSpec(memory_space=pltpu.SMEM)
    sdyn, cdyn, sst, cst = pl.pallas_call(
        _loss_body,
        out_shape=(scal, scal, scal, scal),
        out_specs=(sspec, sspec, sspec, sspec),
    )(d2, y0r, y1r)

    loss_dynamic = (sdyn[0, 0] / cdyn[0, 0]).astype(jnp.float32)
    loss_static = (sst[0, 0] / cst[0, 0]).astype(jnp.float32)
    return (loss_dynamic, loss_static)


# speedup vs baseline: 1.0768x; 1.0768x over previous
import jax
import jax.numpy as jnp
from jax.experimental import pallas as pl
from jax.experimental.pallas import tpu as pltpu

THRES_STATIC = 0.001
THRES_DIST = 0.002

B, N, M = 4, 4096, 4096
MT = 4096


def _fused_body(x_ref, w_ref, y_ref, y8_ref, y1_ref,
                sdyn_ref, cdyn_ref, sst_ref, cst_ref):
    a = x_ref[0]
    yb = y_ref[0]
    aa = jnp.sum(a * a, axis=1, keepdims=True)
    ab = jax.lax.dot_general(
        w_ref[0], y8_ref[0],
        (((1,), (0,)), ((), ())),
        preferred_element_type=jnp.float32,
    )
    t = ab + aa
    tmin = jnp.min(t, axis=0, keepdims=True)
    bb = jnp.sum(yb * yb, axis=0, keepdims=True)
    d2 = jnp.maximum(tmin + bb, 0.0)

    pos = d2 < THRES_DIST
    neg = d2 > THRES_STATIC
    diff = y1_ref[0] - yb
    sq = jnp.sum(diff * diff, axis=0, keepdims=True)
    sel_dyn = jnp.logical_and(pos, sq > 0.0)
    norm = jnp.sqrt(jnp.where(sel_dyn, sq, 1.0))
    z = 0.1 - norm
    sp = jnp.maximum(z, 0.0) + jnp.log1p(jnp.exp(-jnp.abs(z)))
    sdyn = jnp.sum(jnp.where(sel_dyn, sp, 0.0))
    cdyn = jnp.sum(sel_dyn.astype(jnp.float32))
    sel_st = jnp.logical_and(neg, diff > 0.0)
    sst = jnp.sum(jnp.where(sel_st, diff * diff, 0.0))
    cst = jnp.sum(sel_st.astype(jnp.float32))

    @pl.when(jnp.logical_and(pl.program_id(0) == 0, pl.program_id(1) == 0))
    def _():
        sdyn_ref[0, 0] = jnp.float32(0.0)
        cdyn_ref[0, 0] = jnp.float32(0.0)
        sst_ref[0, 0] = jnp.float32(0.0)
        cst_ref[0, 0] = jnp.float32(0.0)

    sdyn_ref[0, 0] += sdyn
    cdyn_ref[0, 0] += cdyn
    sst_ref[0, 0] += sst
    cst_ref[0, 0] += cst


@jax.jit
def kernel(x, y_hat0, y_hat1):
    xp = x[:, 0, :, :3].astype(jnp.float32)
    y0p = y_hat0[:, 0].astype(jnp.float32)
    yt = jnp.transpose(y0p, (0, 2, 1))
    wp = jnp.zeros((B, N, 8), jnp.bfloat16).at[:, :, :3].set(
        (-2.0 * xp).astype(jnp.bfloat16))
    y8 = jnp.zeros((B, 8, M), jnp.bfloat16).at[:, :3, :].set(
        yt.astype(jnp.bfloat16))

    y1t = jnp.transpose(y_hat1[:, 0].astype(jnp.float32), (0, 2, 1))

    scal = jax.ShapeDtypeStruct((1, 1), jnp.float32)
    sspec = pl.BlockSpec(memory_space=pltpu.SMEM)
    sdyn, cdyn, sst, cst = pl.pallas_call(
        _fused_body,
        grid=(B, M // MT),
        in_specs=[
            pl.BlockSpec((1, N, 3), lambda b, m: (b, 0, 0)),
            pl.BlockSpec((1, N, 8), lambda b, m: (b, 0, 0)),
            pl.BlockSpec((1, 3, MT), lambda b, m: (b, 0, m)),
            pl.BlockSpec((1, 8, MT), lambda b, m: (b, 0, m)),
            pl.BlockSpec((1, 3, MT), lambda b, m: (b, 0, m)),
        ],
        out_shape=(scal, scal, scal, scal),
        out_specs=(sspec, sspec, sspec, sspec),
    )(xp, wp, yt, y8, y1t)

    loss_dynamic = (sdyn[0, 0] / cdyn[0, 0]).astype(jnp.float32)
    loss_static = (sst[0, 0] / cst[0, 0]).astype(jnp.float32)
    return (loss_dynamic, loss_static)
